# baseline (speedup 1.0000x reference)
import functools

import jax
import jax.numpy as jnp
from jax import lax
from jax.experimental import pallas as pl
from jax.experimental.pallas import tpu as pltpu
from jax.experimental.pallas import tpu_sc as plsc

EMB_DIM = 32
LANES = 16
NUM_CORES = 2
NUM_SUBCORES = 16
NUM_WORKERS = NUM_CORES * NUM_SUBCORES
BATCH = 16384
BPW = BATCH // NUM_WORKERS
VROW = 128
VTAB = 250000
PASS = 128
NPASS = BPW // PASS


def _sc_body(p1_hbm, p2_hbm, w_hbm, out_hbm, idx1_v, idx2_v, vrow1_v,
             vrow2_v, rows1_v, rows2_v, out_v, sem):
  wid = lax.axis_index("s") * NUM_CORES + lax.axis_index("c")
  base = wid * BPW

  pltpu.sync_copy(p1_hbm.at[pl.ds(base, BPW)], idx1_v)
  pltpu.sync_copy(p2_hbm.at[pl.ds(base, BPW)], idx2_v)

  def pass_body(p, carry):
    pbase = p * PASS

    def vrow_body(j, carry2):
      sl_src = pl.ds(pbase + j * LANES, LANES)
      sl_dst = pl.ds(j * LANES, LANES)
      vrow1_v[sl_dst] = idx1_v[sl_src] >> 2
      vrow2_v[sl_dst] = idx2_v[sl_src] >> 2
      return carry2

    lax.fori_loop(0, PASS // LANES, vrow_body, 0)

    cp1 = pltpu.async_copy(w_hbm.at[vrow1_v], rows1_v, sem)
    cp2 = pltpu.async_copy(w_hbm.at[vrow2_v], rows2_v, sem)
    cp1.wait()
    cp2.wait()

    def chunk_body(c, carry2):
      sl = pl.ds(pbase + c * LANES, LANES)
      i1 = idx1_v[sl]
      i2 = idx2_v[sl]
      col1 = (i1 & 3) << 5
      col2 = (i2 & 3) << 5
      row_ids = lax.broadcasted_iota(jnp.int32, (LANES,), 0) + c * LANES
      acc = jnp.zeros((LANES,), jnp.float32)
      for d in range(EMB_DIM):
        a = plsc.load_gather(rows1_v, [row_ids, col1 + d])
        b = plsc.load_gather(rows2_v, [row_ids, col2 + d])
        acc = acc + a * b
      out_v[sl] = acc
      return carry2

    lax.fori_loop(0, PASS // LANES, chunk_body, 0)
    return carry

  lax.fori_loop(0, NPASS, pass_body, 0)

  pltpu.sync_copy(out_v, out_hbm.at[pl.ds(base, BPW)])


@jax.jit
def _mf_dot(product1, product2, w_view):
  mesh = plsc.VectorSubcoreMesh(core_axis_name="c", subcore_axis_name="s")
  f = pl.kernel(
      _sc_body,
      out_type=jax.ShapeDtypeStruct((BATCH,), jnp.float32),
      mesh=mesh,
      scratch_types=[
          pltpu.VMEM((BPW,), jnp.int32),
          pltpu.VMEM((BPW,), jnp.int32),
          pltpu.VMEM((PASS,), jnp.int32),
          pltpu.VMEM((PASS,), jnp.int32),
          pltpu.VMEM((PASS, VROW), jnp.float32),
          pltpu.VMEM((PASS, VROW), jnp.float32),
          pltpu.VMEM((BPW,), jnp.float32),
          pltpu.SemaphoreType.DMA,
      ],
      compiler_params=pltpu.CompilerParams(needs_layout_passes=False,
                                           use_tc_tiling_on_sc=True),
  )
  return f(product1, product2, w_view)


def kernel(product1, product2, embedding_weight):
  w_view = embedding_weight.reshape(VTAB, VROW)
  return _mf_dot(product1.astype(jnp.int32), product2.astype(jnp.int32),
                 w_view)
